# baseline (speedup 1.0000x reference)
import functools

import jax
import jax.numpy as jnp
from jax import lax
from jax.experimental import pallas as pl
from jax.experimental.pallas import tpu as pltpu
from jax.experimental.pallas import tpu_sc as plsc

_KT = 512
_QT = 512

_NC = 2
_NS = 16
_NW = _NC * _NS


def _argmax_body(xin_ref, rin_ref, biask_ref, maskq_ref, ind_ref, bestv, besti):
    kt, qt = _KT, _QT
    b = pl.program_id(0)
    q = pl.program_id(1)
    k = pl.program_id(2)
    nk = pl.num_programs(2)
    hw = nk * kt

    x = xin_ref[0]
    r = rin_ref[0]
    nrm = jnp.sqrt(jnp.sum(x * x, axis=0, keepdims=True)) + 1e-8
    xk = x / nrm
    sim = lax.dot_general(
        xk, r, (((0,), (0,)), ((), ())),
        preferred_element_type=jnp.float32,
    )
    sim = sim + biask_ref[...]

    tmax = jnp.max(sim, axis=0, keepdims=True)
    kio = lax.broadcasted_iota(jnp.int32, (kt, qt), 0)
    cand = jnp.where(sim == tmax, kio, jnp.int32(kt))
    targ = jnp.min(cand, axis=0, keepdims=True) + k * kt

    @pl.when(k == 0)
    def _init():
        bestv[...] = tmax
        besti[...] = targ

    @pl.when(k > 0)
    def _update():
        upd = tmax > bestv[...]
        bestv[...] = jnp.where(upd, tmax, bestv[...])
        besti[...] = jnp.where(upd, targ, besti[...])

    @pl.when(k == nk - 1)
    def _emit():
        mq = maskq_ref[...]
        qio = lax.broadcasted_iota(jnp.int32, (1, qt), 1) + q * qt
        ind_ref[0] = jnp.where(mq > 0, besti[...], qio) + b * hw


def _argmax_indices(xin, rin, biask, maskq):
    B, C, HW = xin.shape
    nq, nk = HW // _QT, HW // _KT
    return pl.pallas_call(
        _argmax_body,
        grid=(B, nq, nk),
        in_specs=[
            pl.BlockSpec((1, C, _KT), lambda b, q, k: (b, 0, k)),
            pl.BlockSpec((1, C, _QT), lambda b, q, k: (b, 0, q)),
            pl.BlockSpec((_KT, 1), lambda b, q, k: (k, 0)),
            pl.BlockSpec((1, _QT), lambda b, q, k: (0, q)),
        ],
        out_specs=pl.BlockSpec((1, 1, _QT), lambda b, q, k: (b, 0, q)),
        out_shape=jax.ShapeDtypeStruct((B, 1, HW), jnp.int32),
        scratch_shapes=[
            pltpu.VMEM((1, _QT), jnp.float32),
            pltpu.VMEM((1, _QT), jnp.int32),
        ],
        compiler_params=pltpu.CompilerParams(
            dimension_semantics=("parallel", "parallel", "arbitrary"),
        ),
    )(xin, rin, biask, maskq)


def _sc_gather(table, idx):
    nrows, C = table.shape
    rows_per_w = nrows // _NW
    chunk = min(rows_per_w, 256)

    mesh = plsc.VectorSubcoreMesh(core_axis_name="c", subcore_axis_name="s")

    @functools.partial(
        pl.kernel,
        mesh=mesh,
        out_type=jax.ShapeDtypeStruct((nrows, C), jnp.float32),
        scratch_types=[
            pltpu.VMEM((rows_per_w,), jnp.int32),
            pltpu.VMEM((chunk, C), jnp.float32),
            pltpu.SemaphoreType.DMA,
        ],
    )
    def gather_k(table_hbm, idx_hbm, out_hbm, idx_v, buf, sem):
        wid = lax.axis_index("s") * _NC + lax.axis_index("c")
        base = wid * rows_per_w
        pltpu.sync_copy(idx_hbm.at[pl.ds(base, rows_per_w)], idx_v)
        for c in range(rows_per_w // chunk):
            src = table_hbm.at[idx_v.at[pl.ds(c * chunk, chunk)]]
            pltpu.async_copy(src, buf, sem).wait()
            pltpu.sync_copy(buf, out_hbm.at[pl.ds(base + c * chunk, chunk)])

    return gather_k(table, idx)


def kernel(input, ref, mask):
    B, C, H, W = input.shape
    HW = H * W
    xin = input.reshape(B, C, HW)
    rin = ref.reshape(B, C, HW)
    mflat = mask.reshape(HW)
    biask = (mflat.astype(jnp.float32) * jnp.float32(-1e30)).reshape(HW, 1)
    maskq = mflat.reshape(1, HW).astype(jnp.int32)

    ind = _argmax_indices(xin, rin, biask, maskq)
    table = xin.transpose(0, 2, 1).reshape(B * HW, C)
    rows = _sc_gather(table, ind.reshape(B * HW))
    return rows.reshape(B, HW, C).transpose(0, 2, 1).reshape(B, C, H, W)


# speedup vs baseline: 1.4264x; 1.4264x over previous
import functools

import jax
import jax.numpy as jnp
from jax import lax
from jax.experimental import pallas as pl
from jax.experimental.pallas import tpu as pltpu
from jax.experimental.pallas import tpu_sc as plsc

_KT = 512
_QT = 512
_SUB = 8

_NC = 2
_NS = 16
_NW = _NC * _NS


def _argmax_body(xin_ref, rin_ref, maskk_ref, maskq_ref, ind_ref,
                 xk_s, bestv, besti):
    kt, qt = _KT, _QT
    b = pl.program_id(0)
    k = pl.program_id(1)
    q = pl.program_id(2)
    nk = pl.num_programs(1)
    hw = nk * kt
    qsl = pl.ds(q * qt, qt)

    @pl.when(q == 0)
    def _prep_bank():
        x = xin_ref[0, :, pl.ds(k * kt, kt)]
        nrm = jnp.sqrt(jnp.sum(x * x, axis=0, keepdims=True)) + 1e-8
        xk_s[...] = x / nrm

    r = rin_ref[0, :, qsl]
    sim = lax.dot_general(
        xk_s[...], r, (((0,), (0,)), ((), ())),
        preferred_element_type=jnp.float32,
    )

    mk = maskk_ref[pl.ds(k * kt, kt), :] > 0
    ninf = jnp.float32(-jnp.inf)

    rv = jnp.where(mk[0:_SUB], ninf, sim[0:_SUB])
    ri = lax.broadcasted_iota(jnp.int32, (_SUB, qt), 0) + k * kt
    for j in range(1, kt // _SUB):
        lo = j * _SUB
        v = jnp.where(mk[lo:lo + _SUB], ninf, sim[lo:lo + _SUB])
        i = lax.broadcasted_iota(jnp.int32, (_SUB, qt), 0) + (k * kt + lo)
        upd = v > rv
        rv = jnp.where(upd, v, rv)
        ri = jnp.where(upd, i, ri)

    tmax = jnp.max(rv, axis=0, keepdims=True)
    cand = jnp.where(rv == tmax, ri, jnp.int32(2**30))
    targ = jnp.min(cand, axis=0, keepdims=True)

    @pl.when(k == 0)
    def _init():
        bestv[:, qsl] = tmax
        besti[:, qsl] = targ

    @pl.when(k > 0)
    def _update():
        bv = bestv[:, qsl]
        bi = besti[:, qsl]
        upd = tmax > bv
        bestv[:, qsl] = jnp.where(upd, tmax, bv)
        besti[:, qsl] = jnp.where(upd, targ, bi)

    @pl.when(k == nk - 1)
    def _emit():
        mq = maskq_ref[:, qsl]
        qio = lax.broadcasted_iota(jnp.int32, (1, qt), 1) + q * qt
        ind_ref[0, :, qsl] = jnp.where(mq > 0, besti[:, qsl], qio) + b * hw


def _argmax_indices(xin, rin, maskk, maskq):
    B, C, HW = xin.shape
    nq, nk = HW // _QT, HW // _KT
    return pl.pallas_call(
        _argmax_body,
        grid=(B, nk, nq),
        in_specs=[
            pl.BlockSpec((1, C, HW), lambda b, k, q: (b, 0, 0)),
            pl.BlockSpec((1, C, HW), lambda b, k, q: (b, 0, 0)),
            pl.BlockSpec((HW, 1), lambda b, k, q: (0, 0)),
            pl.BlockSpec((1, HW), lambda b, k, q: (0, 0)),
        ],
        out_specs=pl.BlockSpec((1, 1, HW), lambda b, k, q: (b, 0, 0)),
        out_shape=jax.ShapeDtypeStruct((B, 1, HW), jnp.int32),
        scratch_shapes=[
            pltpu.VMEM((C, _KT), jnp.float32),
            pltpu.VMEM((1, HW), jnp.float32),
            pltpu.VMEM((1, HW), jnp.int32),
        ],
        compiler_params=pltpu.CompilerParams(
            dimension_semantics=("parallel", "arbitrary", "arbitrary"),
        ),
    )(xin, rin, maskk, maskq)


def _sc_gather(table, idx):
    nrows, C = table.shape
    rows_per_w = nrows // _NW
    chunk = min(rows_per_w, 256)

    mesh = plsc.VectorSubcoreMesh(core_axis_name="c", subcore_axis_name="s")

    @functools.partial(
        pl.kernel,
        mesh=mesh,
        out_type=jax.ShapeDtypeStruct((nrows, C), jnp.float32),
        scratch_types=[
            pltpu.VMEM((rows_per_w,), jnp.int32),
            pltpu.VMEM((chunk, C), jnp.float32),
            pltpu.SemaphoreType.DMA,
        ],
    )
    def gather_k(table_hbm, idx_hbm, out_hbm, idx_v, buf, sem):
        wid = lax.axis_index("s") * _NC + lax.axis_index("c")
        base = wid * rows_per_w
        pltpu.sync_copy(idx_hbm.at[pl.ds(base, rows_per_w)], idx_v)
        for c in range(rows_per_w // chunk):
            src = table_hbm.at[idx_v.at[pl.ds(c * chunk, chunk)]]
            pltpu.async_copy(src, buf, sem).wait()
            pltpu.sync_copy(buf, out_hbm.at[pl.ds(base + c * chunk, chunk)])

    return gather_k(table, idx)


def kernel(input, ref, mask):
    B, C, H, W = input.shape
    HW = H * W
    xin = input.reshape(B, C, HW)
    rin = ref.reshape(B, C, HW)
    mflat = mask.reshape(HW).astype(jnp.int32)
    maskk = mflat.reshape(HW, 1)
    maskq = mflat.reshape(1, HW)

    ind = _argmax_indices(xin, rin, maskk, maskq)
    table = xin.transpose(0, 2, 1).reshape(B * HW, C)
    rows = _sc_gather(table, ind.reshape(B * HW))
    return rows.reshape(B, HW, C).transpose(0, 2, 1).reshape(B, C, H, W)


# speedup vs baseline: 1.8515x; 1.2981x over previous
import functools

import jax
import jax.numpy as jnp
from jax import lax
from jax.experimental import pallas as pl
from jax.experimental.pallas import tpu as pltpu
from jax.experimental.pallas import tpu_sc as plsc

_KT = 512
_QT = 1024
_SUB = 8

_NC = 2
_NS = 16
_NW = _NC * _NS


def _argmax_body(xin_ref, rin_ref, maskk_ref, maskq_ref, ind_ref,
                 xk_s, bestv, besti):
    kt, qt = _KT, _QT
    b = pl.program_id(0)
    k = pl.program_id(1)
    q = pl.program_id(2)
    nk = pl.num_programs(1)
    hw = nk * kt
    qsl = pl.ds(q * qt, qt)

    @pl.when(q == 0)
    def _prep_bank():
        x = xin_ref[0, :, pl.ds(k * kt, kt)]
        nrm = jnp.sqrt(jnp.sum(x * x, axis=0, keepdims=True)) + 1e-8
        xk_s[...] = x / nrm

    r = rin_ref[0, :, qsl]
    sim = lax.dot_general(
        xk_s[...], r, (((0,), (0,)), ((), ())),
        preferred_element_type=jnp.float32,
    )

    mk = maskk_ref[pl.ds(k * kt, kt), :] > 0
    ninf = jnp.float32(-jnp.inf)

    rv = jnp.where(mk[0:_SUB], ninf, sim[0:_SUB])
    ri = lax.broadcasted_iota(jnp.int32, (_SUB, qt), 0) + k * kt
    for j in range(1, kt // _SUB):
        lo = j * _SUB
        v = jnp.where(mk[lo:lo + _SUB], ninf, sim[lo:lo + _SUB])
        i = lax.broadcasted_iota(jnp.int32, (_SUB, qt), 0) + (k * kt + lo)
        upd = v > rv
        rv = jnp.where(upd, v, rv)
        ri = jnp.where(upd, i, ri)

    tmax = jnp.max(rv, axis=0, keepdims=True)
    cand = jnp.where(rv == tmax, ri, jnp.int32(2**30))
    targ = jnp.min(cand, axis=0, keepdims=True)

    @pl.when(k == 0)
    def _init():
        bestv[:, qsl] = tmax
        besti[:, qsl] = targ

    @pl.when(k > 0)
    def _update():
        bv = bestv[:, qsl]
        bi = besti[:, qsl]
        upd = tmax > bv
        bestv[:, qsl] = jnp.where(upd, tmax, bv)
        besti[:, qsl] = jnp.where(upd, targ, bi)

    @pl.when(k == nk - 1)
    def _emit():
        mq = maskq_ref[:, qsl]
        qio = lax.broadcasted_iota(jnp.int32, (1, qt), 1) + q * qt
        ind_ref[0, :, qsl] = jnp.where(mq > 0, besti[:, qsl], qio) + b * hw


def _argmax_indices(xin, rin, maskk, maskq):
    B, C, HW = xin.shape
    nq, nk = HW // _QT, HW // _KT
    return pl.pallas_call(
        _argmax_body,
        grid=(B, nk, nq),
        in_specs=[
            pl.BlockSpec((1, C, HW), lambda b, k, q: (b, 0, 0)),
            pl.BlockSpec((1, C, HW), lambda b, k, q: (b, 0, 0)),
            pl.BlockSpec((HW, 1), lambda b, k, q: (0, 0)),
            pl.BlockSpec((1, HW), lambda b, k, q: (0, 0)),
        ],
        out_specs=pl.BlockSpec((1, 1, HW), lambda b, k, q: (b, 0, 0)),
        out_shape=jax.ShapeDtypeStruct((B, 1, HW), jnp.int32),
        scratch_shapes=[
            pltpu.VMEM((C, _KT), jnp.float32),
            pltpu.VMEM((1, HW), jnp.float32),
            pltpu.VMEM((1, HW), jnp.int32),
        ],
        compiler_params=pltpu.CompilerParams(
            dimension_semantics=("parallel", "arbitrary", "arbitrary"),
        ),
    )(xin, rin, maskk, maskq)


def _sc_gather(table, idx):
    nrows, C = table.shape
    rows_per_w = nrows // _NW
    chunk = min(rows_per_w, 256)

    mesh = plsc.VectorSubcoreMesh(core_axis_name="c", subcore_axis_name="s")

    @functools.partial(
        pl.kernel,
        mesh=mesh,
        out_type=jax.ShapeDtypeStruct((nrows, C), jnp.float32),
        scratch_types=[
            pltpu.VMEM((rows_per_w,), jnp.int32),
            pltpu.VMEM((chunk, C), jnp.float32),
            pltpu.SemaphoreType.DMA,
        ],
    )
    def gather_k(table_hbm, idx_hbm, out_hbm, idx_v, buf, sem):
        wid = lax.axis_index("s") * _NC + lax.axis_index("c")
        base = wid * rows_per_w
        pltpu.sync_copy(idx_hbm.at[pl.ds(base, rows_per_w)], idx_v)
        for c in range(rows_per_w // chunk):
            src = table_hbm.at[idx_v.at[pl.ds(c * chunk, chunk)]]
            pltpu.async_copy(src, buf, sem).wait()
            pltpu.sync_copy(buf, out_hbm.at[pl.ds(base + c * chunk, chunk)])

    return gather_k(table, idx)


def kernel(input, ref, mask):
    B, C, H, W = input.shape
    HW = H * W
    xin = input.reshape(B, C, HW)
    rin = ref.reshape(B, C, HW)
    mflat = mask.reshape(HW).astype(jnp.int32)
    maskk = mflat.reshape(HW, 1)
    maskq = mflat.reshape(1, HW)

    ind = _argmax_indices(xin, rin, maskk, maskq)
    table = xin.transpose(0, 2, 1).reshape(B * HW, C)
    rows = _sc_gather(table, ind.reshape(B * HW))
    return rows.reshape(B, HW, C).transpose(0, 2, 1).reshape(B, C, H, W)


# speedup vs baseline: 2.1223x; 1.1462x over previous
import functools

import jax
import jax.numpy as jnp
from jax import lax
from jax.experimental import pallas as pl
from jax.experimental.pallas import tpu as pltpu
from jax.experimental.pallas import tpu_sc as plsc

_KT = 512
_QT = 2048
_SUB = 8

_NC = 2
_NS = 16
_NW = _NC * _NS


def _argmax_body(xin_ref, rin_ref, maskk_ref, maskq_ref, ind_ref,
                 xk_s, bestv, besti):
    kt, qt = _KT, _QT
    b = pl.program_id(0)
    k = pl.program_id(1)
    q = pl.program_id(2)
    nk = pl.num_programs(1)
    hw = nk * kt
    qsl = pl.ds(q * qt, qt)

    @pl.when(q == 0)
    def _prep_bank():
        x = xin_ref[0, :, pl.ds(k * kt, kt)]
        nrm = jnp.sqrt(jnp.sum(x * x, axis=0, keepdims=True)) + 1e-8
        xk_s[...] = x / nrm

    r = rin_ref[0, :, qsl]
    sim = lax.dot_general(
        xk_s[...], r, (((0,), (0,)), ((), ())),
        preferred_element_type=jnp.float32,
    )

    mk = maskk_ref[pl.ds(k * kt, kt), :] > 0
    ninf = jnp.float32(-jnp.inf)

    rv = jnp.where(mk[0:_SUB], ninf, sim[0:_SUB])
    ri = lax.broadcasted_iota(jnp.int32, (_SUB, qt), 0) + k * kt
    for j in range(1, kt // _SUB):
        lo = j * _SUB
        v = jnp.where(mk[lo:lo + _SUB], ninf, sim[lo:lo + _SUB])
        i = lax.broadcasted_iota(jnp.int32, (_SUB, qt), 0) + (k * kt + lo)
        upd = v > rv
        rv = jnp.where(upd, v, rv)
        ri = jnp.where(upd, i, ri)

    tmax = jnp.max(rv, axis=0, keepdims=True)
    cand = jnp.where(rv == tmax, ri, jnp.int32(2**30))
    targ = jnp.min(cand, axis=0, keepdims=True)

    @pl.when(k == 0)
    def _init():
        bestv[:, qsl] = tmax
        besti[:, qsl] = targ

    @pl.when(k > 0)
    def _update():
        bv = bestv[:, qsl]
        bi = besti[:, qsl]
        upd = tmax > bv
        bestv[:, qsl] = jnp.where(upd, tmax, bv)
        besti[:, qsl] = jnp.where(upd, targ, bi)

    @pl.when(k == nk - 1)
    def _emit():
        mq = maskq_ref[:, qsl]
        qio = lax.broadcasted_iota(jnp.int32, (1, qt), 1) + q * qt
        ind_ref[0, :, qsl] = jnp.where(mq > 0, besti[:, qsl], qio) + b * hw


def _argmax_indices(xin, rin, maskk, maskq):
    B, C, HW = xin.shape
    nq, nk = HW // _QT, HW // _KT
    return pl.pallas_call(
        _argmax_body,
        grid=(B, nk, nq),
        in_specs=[
            pl.BlockSpec((1, C, HW), lambda b, k, q: (b, 0, 0)),
            pl.BlockSpec((1, C, HW), lambda b, k, q: (b, 0, 0)),
            pl.BlockSpec((HW, 1), lambda b, k, q: (0, 0)),
            pl.BlockSpec((1, HW), lambda b, k, q: (0, 0)),
        ],
        out_specs=pl.BlockSpec((1, 1, HW), lambda b, k, q: (b, 0, 0)),
        out_shape=jax.ShapeDtypeStruct((B, 1, HW), jnp.int32),
        scratch_shapes=[
            pltpu.VMEM((C, _KT), jnp.float32),
            pltpu.VMEM((1, HW), jnp.float32),
            pltpu.VMEM((1, HW), jnp.int32),
        ],
        compiler_params=pltpu.CompilerParams(
            dimension_semantics=("parallel", "arbitrary", "arbitrary"),
        ),
    )(xin, rin, maskk, maskq)


def _sc_gather(table, idx):
    nrows, C = table.shape
    rows_per_w = nrows // _NW
    chunk = min(rows_per_w, 256)

    mesh = plsc.VectorSubcoreMesh(core_axis_name="c", subcore_axis_name="s")

    @functools.partial(
        pl.kernel,
        mesh=mesh,
        out_type=jax.ShapeDtypeStruct((nrows, C), jnp.float32),
        scratch_types=[
            pltpu.VMEM((rows_per_w,), jnp.int32),
            pltpu.VMEM((chunk, C), jnp.float32),
            pltpu.SemaphoreType.DMA,
        ],
    )
    def gather_k(table_hbm, idx_hbm, out_hbm, idx_v, buf, sem):
        wid = lax.axis_index("s") * _NC + lax.axis_index("c")
        base = wid * rows_per_w
        pltpu.sync_copy(idx_hbm.at[pl.ds(base, rows_per_w)], idx_v)
        for c in range(rows_per_w // chunk):
            src = table_hbm.at[idx_v.at[pl.ds(c * chunk, chunk)]]
            pltpu.async_copy(src, buf, sem).wait()
            pltpu.sync_copy(buf, out_hbm.at[pl.ds(base + c * chunk, chunk)])

    return gather_k(table, idx)


def kernel(input, ref, mask):
    B, C, H, W = input.shape
    HW = H * W
    xin = input.reshape(B, C, HW)
    rin = ref.reshape(B, C, HW)
    mflat = mask.reshape(HW).astype(jnp.int32)
    maskk = mflat.reshape(HW, 1)
    maskq = mflat.reshape(1, HW)

    ind = _argmax_indices(xin, rin, maskk, maskq)
    table = xin.transpose(0, 2, 1).reshape(B * HW, C)
    rows = _sc_gather(table, ind.reshape(B * HW))
    return rows.reshape(B, HW, C).transpose(0, 2, 1).reshape(B, C, H, W)


# speedup vs baseline: 2.3734x; 1.1183x over previous
import functools

import jax
import jax.numpy as jnp
from jax import lax
from jax.experimental import pallas as pl
from jax.experimental.pallas import tpu as pltpu
from jax.experimental.pallas import tpu_sc as plsc

_KT = 512
_QT = 4096
_SUB = 8

_NC = 2
_NS = 16
_NW = _NC * _NS


def _argmax_body(xin_ref, rin_ref, maskk_ref, maskq_ref, ind_ref,
                 xk_s, bestv, besti):
    kt, qt = _KT, _QT
    b = pl.program_id(0)
    k = pl.program_id(1)
    q = pl.program_id(2)
    nk = pl.num_programs(1)
    hw = nk * kt
    qsl = pl.ds(q * qt, qt)

    @pl.when(q == 0)
    def _prep_bank():
        x = xin_ref[0, :, pl.ds(k * kt, kt)]
        nrm = jnp.sqrt(jnp.sum(x * x, axis=0, keepdims=True)) + 1e-8
        xk_s[...] = x / nrm

    r = rin_ref[0, :, qsl]
    sim = lax.dot_general(
        xk_s[...], r, (((0,), (0,)), ((), ())),
        preferred_element_type=jnp.float32,
    )

    mk = maskk_ref[pl.ds(k * kt, kt), :] > 0
    ninf = jnp.float32(-jnp.inf)

    rv = jnp.where(mk[0:_SUB], ninf, sim[0:_SUB])
    ri = lax.broadcasted_iota(jnp.int32, (_SUB, qt), 0) + k * kt
    for j in range(1, kt // _SUB):
        lo = j * _SUB
        v = jnp.where(mk[lo:lo + _SUB], ninf, sim[lo:lo + _SUB])
        i = lax.broadcasted_iota(jnp.int32, (_SUB, qt), 0) + (k * kt + lo)
        upd = v > rv
        rv = jnp.where(upd, v, rv)
        ri = jnp.where(upd, i, ri)

    tmax = jnp.max(rv, axis=0, keepdims=True)
    cand = jnp.where(rv == tmax, ri, jnp.int32(2**30))
    targ = jnp.min(cand, axis=0, keepdims=True)

    @pl.when(k == 0)
    def _init():
        bestv[:, qsl] = tmax
        besti[:, qsl] = targ

    @pl.when(k > 0)
    def _update():
        bv = bestv[:, qsl]
        bi = besti[:, qsl]
        upd = tmax > bv
        bestv[:, qsl] = jnp.where(upd, tmax, bv)
        besti[:, qsl] = jnp.where(upd, targ, bi)

    @pl.when(k == nk - 1)
    def _emit():
        mq = maskq_ref[:, qsl]
        qio = lax.broadcasted_iota(jnp.int32, (1, qt), 1) + q * qt
        ind_ref[0, :, qsl] = jnp.where(mq > 0, besti[:, qsl], qio) + b * hw


def _argmax_indices(xin, rin, maskk, maskq):
    B, C, HW = xin.shape
    nq, nk = HW // _QT, HW // _KT
    return pl.pallas_call(
        _argmax_body,
        grid=(B, nk, nq),
        in_specs=[
            pl.BlockSpec((1, C, HW), lambda b, k, q: (b, 0, 0)),
            pl.BlockSpec((1, C, HW), lambda b, k, q: (b, 0, 0)),
            pl.BlockSpec((HW, 1), lambda b, k, q: (0, 0)),
            pl.BlockSpec((1, HW), lambda b, k, q: (0, 0)),
        ],
        out_specs=pl.BlockSpec((1, 1, HW), lambda b, k, q: (b, 0, 0)),
        out_shape=jax.ShapeDtypeStruct((B, 1, HW), jnp.int32),
        scratch_shapes=[
            pltpu.VMEM((C, _KT), jnp.float32),
            pltpu.VMEM((1, HW), jnp.float32),
            pltpu.VMEM((1, HW), jnp.int32),
        ],
        compiler_params=pltpu.CompilerParams(
            dimension_semantics=("parallel", "arbitrary", "arbitrary"),
        ),
    )(xin, rin, maskk, maskq)


def _sc_gather(table, idx):
    nrows, C = table.shape
    rows_per_w = nrows // _NW
    chunk = min(rows_per_w, 256)

    mesh = plsc.VectorSubcoreMesh(core_axis_name="c", subcore_axis_name="s")

    @functools.partial(
        pl.kernel,
        mesh=mesh,
        out_type=jax.ShapeDtypeStruct((nrows, C), jnp.float32),
        scratch_types=[
            pltpu.VMEM((rows_per_w,), jnp.int32),
            pltpu.VMEM((chunk, C), jnp.float32),
            pltpu.SemaphoreType.DMA,
        ],
    )
    def gather_k(table_hbm, idx_hbm, out_hbm, idx_v, buf, sem):
        wid = lax.axis_index("s") * _NC + lax.axis_index("c")
        base = wid * rows_per_w
        pltpu.sync_copy(idx_hbm.at[pl.ds(base, rows_per_w)], idx_v)
        for c in range(rows_per_w // chunk):
            src = table_hbm.at[idx_v.at[pl.ds(c * chunk, chunk)]]
            pltpu.async_copy(src, buf, sem).wait()
            pltpu.sync_copy(buf, out_hbm.at[pl.ds(base + c * chunk, chunk)])

    return gather_k(table, idx)


def kernel(input, ref, mask):
    B, C, H, W = input.shape
    HW = H * W
    xin = input.reshape(B, C, HW)
    rin = ref.reshape(B, C, HW)
    mflat = mask.reshape(HW).astype(jnp.int32)
    maskk = mflat.reshape(HW, 1)
    maskq = mflat.reshape(1, HW)

    ind = _argmax_indices(xin, rin, maskk, maskq)
    table = xin.transpose(0, 2, 1).reshape(B * HW, C)
    rows = _sc_gather(table, ind.reshape(B * HW))
    return rows.reshape(B, HW, C).transpose(0, 2, 1).reshape(B, C, H, W)


# speedup vs baseline: 2.5280x; 1.0651x over previous
import functools

import jax
import jax.numpy as jnp
from jax import lax
from jax.experimental import pallas as pl
from jax.experimental.pallas import tpu as pltpu
from jax.experimental.pallas import tpu_sc as plsc

_KT = 1024
_QT = 4096
_SUB = 8

_NC = 2
_NS = 16
_NW = _NC * _NS


def _argmax_body(xin_ref, rin_ref, maskk_ref, maskq_ref, ind_ref,
                 xk_s, bestv, besti):
    kt, qt = _KT, _QT
    b = pl.program_id(0)
    k = pl.program_id(1)
    q = pl.program_id(2)
    nk = pl.num_programs(1)
    hw = nk * kt
    qsl = pl.ds(q * qt, qt)

    @pl.when(q == 0)
    def _prep_bank():
        x = xin_ref[0, :, pl.ds(k * kt, kt)]
        nrm = jnp.sqrt(jnp.sum(x * x, axis=0, keepdims=True)) + 1e-8
        xk_s[...] = x / nrm

    r = rin_ref[0, :, qsl]
    sim = lax.dot_general(
        xk_s[...], r, (((0,), (0,)), ((), ())),
        preferred_element_type=jnp.float32,
    )

    mk = maskk_ref[pl.ds(k * kt, kt), :] > 0
    ninf = jnp.float32(-jnp.inf)

    rv = jnp.where(mk[0:_SUB], ninf, sim[0:_SUB])
    ri = lax.broadcasted_iota(jnp.int32, (_SUB, qt), 0) + k * kt
    for j in range(1, kt // _SUB):
        lo = j * _SUB
        v = jnp.where(mk[lo:lo + _SUB], ninf, sim[lo:lo + _SUB])
        i = lax.broadcasted_iota(jnp.int32, (_SUB, qt), 0) + (k * kt + lo)
        upd = v > rv
        rv = jnp.where(upd, v, rv)
        ri = jnp.where(upd, i, ri)

    tmax = jnp.max(rv, axis=0, keepdims=True)
    cand = jnp.where(rv == tmax, ri, jnp.int32(2**30))
    targ = jnp.min(cand, axis=0, keepdims=True)

    @pl.when(k == 0)
    def _init():
        bestv[:, qsl] = tmax
        besti[:, qsl] = targ

    @pl.when(k > 0)
    def _update():
        bv = bestv[:, qsl]
        bi = besti[:, qsl]
        upd = tmax > bv
        bestv[:, qsl] = jnp.where(upd, tmax, bv)
        besti[:, qsl] = jnp.where(upd, targ, bi)

    @pl.when(k == nk - 1)
    def _emit():
        mq = maskq_ref[:, qsl]
        qio = lax.broadcasted_iota(jnp.int32, (1, qt), 1) + q * qt
        ind_ref[0, :, qsl] = jnp.where(mq > 0, besti[:, qsl], qio) + b * hw


def _argmax_indices(xin, rin, maskk, maskq):
    B, C, HW = xin.shape
    nq, nk = HW // _QT, HW // _KT
    return pl.pallas_call(
        _argmax_body,
        grid=(B, nk, nq),
        in_specs=[
            pl.BlockSpec((1, C, HW), lambda b, k, q: (b, 0, 0)),
            pl.BlockSpec((1, C, HW), lambda b, k, q: (b, 0, 0)),
            pl.BlockSpec((HW, 1), lambda b, k, q: (0, 0)),
            pl.BlockSpec((1, HW), lambda b, k, q: (0, 0)),
        ],
        out_specs=pl.BlockSpec((1, 1, HW), lambda b, k, q: (b, 0, 0)),
        out_shape=jax.ShapeDtypeStruct((B, 1, HW), jnp.int32),
        scratch_shapes=[
            pltpu.VMEM((C, _KT), jnp.float32),
            pltpu.VMEM((1, HW), jnp.float32),
            pltpu.VMEM((1, HW), jnp.int32),
        ],
        compiler_params=pltpu.CompilerParams(
            dimension_semantics=("parallel", "arbitrary", "arbitrary"),
        ),
    )(xin, rin, maskk, maskq)


def _sc_gather(table, idx):
    nrows, C = table.shape
    rows_per_w = nrows // _NW
    chunk = min(rows_per_w, 256)

    mesh = plsc.VectorSubcoreMesh(core_axis_name="c", subcore_axis_name="s")

    @functools.partial(
        pl.kernel,
        mesh=mesh,
        out_type=jax.ShapeDtypeStruct((nrows, C), jnp.float32),
        scratch_types=[
            pltpu.VMEM((rows_per_w,), jnp.int32),
            pltpu.VMEM((chunk, C), jnp.float32),
            pltpu.SemaphoreType.DMA,
        ],
    )
    def gather_k(table_hbm, idx_hbm, out_hbm, idx_v, buf, sem):
        wid = lax.axis_index("s") * _NC + lax.axis_index("c")
        base = wid * rows_per_w
        pltpu.sync_copy(idx_hbm.at[pl.ds(base, rows_per_w)], idx_v)
        for c in range(rows_per_w // chunk):
            src = table_hbm.at[idx_v.at[pl.ds(c * chunk, chunk)]]
            pltpu.async_copy(src, buf, sem).wait()
            pltpu.sync_copy(buf, out_hbm.at[pl.ds(base + c * chunk, chunk)])

    return gather_k(table, idx)


def kernel(input, ref, mask):
    B, C, H, W = input.shape
    HW = H * W
    xin = input.reshape(B, C, HW)
    rin = ref.reshape(B, C, HW)
    mflat = mask.reshape(HW).astype(jnp.int32)
    maskk = mflat.reshape(HW, 1)
    maskq = mflat.reshape(1, HW)

    ind = _argmax_indices(xin, rin, maskk, maskq)
    table = xin.transpose(0, 2, 1).reshape(B * HW, C)
    rows = _sc_gather(table, ind.reshape(B * HW))
    return rows.reshape(B, HW, C).transpose(0, 2, 1).reshape(B, C, H, W)


# speedup vs baseline: 2.6261x; 1.0388x over previous
import functools

import jax
import jax.numpy as jnp
from jax import lax
from jax.experimental import pallas as pl
from jax.experimental.pallas import tpu as pltpu
from jax.experimental.pallas import tpu_sc as plsc

_KT = 2048
_QT = 4096
_SUB = 8

_NC = 2
_NS = 16
_NW = _NC * _NS


def _argmax_body(xin_ref, rin_ref, maskk_ref, maskq_ref, ind_ref,
                 xk_s, bestv, besti):
    kt, qt = _KT, _QT
    b = pl.program_id(0)
    k = pl.program_id(1)
    q = pl.program_id(2)
    nk = pl.num_programs(1)
    hw = nk * kt
    qsl = pl.ds(q * qt, qt)

    @pl.when(q == 0)
    def _prep_bank():
        x = xin_ref[0, :, pl.ds(k * kt, kt)]
        nrm = jnp.sqrt(jnp.sum(x * x, axis=0, keepdims=True)) + 1e-8
        xk_s[...] = x / nrm

    r = rin_ref[0, :, qsl]
    sim = lax.dot_general(
        xk_s[...], r, (((0,), (0,)), ((), ())),
        preferred_element_type=jnp.float32,
    )

    mk = maskk_ref[pl.ds(k * kt, kt), :] > 0
    ninf = jnp.float32(-jnp.inf)

    rv = jnp.where(mk[0:_SUB], ninf, sim[0:_SUB])
    ri = lax.broadcasted_iota(jnp.int32, (_SUB, qt), 0) + k * kt
    for j in range(1, kt // _SUB):
        lo = j * _SUB
        v = jnp.where(mk[lo:lo + _SUB], ninf, sim[lo:lo + _SUB])
        i = lax.broadcasted_iota(jnp.int32, (_SUB, qt), 0) + (k * kt + lo)
        upd = v > rv
        rv = jnp.where(upd, v, rv)
        ri = jnp.where(upd, i, ri)

    tmax = jnp.max(rv, axis=0, keepdims=True)
    cand = jnp.where(rv == tmax, ri, jnp.int32(2**30))
    targ = jnp.min(cand, axis=0, keepdims=True)

    @pl.when(k == 0)
    def _init():
        bestv[:, qsl] = tmax
        besti[:, qsl] = targ

    @pl.when(k > 0)
    def _update():
        bv = bestv[:, qsl]
        bi = besti[:, qsl]
        upd = tmax > bv
        bestv[:, qsl] = jnp.where(upd, tmax, bv)
        besti[:, qsl] = jnp.where(upd, targ, bi)

    @pl.when(k == nk - 1)
    def _emit():
        mq = maskq_ref[:, qsl]
        qio = lax.broadcasted_iota(jnp.int32, (1, qt), 1) + q * qt
        ind_ref[0, :, qsl] = jnp.where(mq > 0, besti[:, qsl], qio) + b * hw


def _argmax_indices(xin, rin, maskk, maskq):
    B, C, HW = xin.shape
    nq, nk = HW // _QT, HW // _KT
    return pl.pallas_call(
        _argmax_body,
        grid=(B, nk, nq),
        in_specs=[
            pl.BlockSpec((1, C, HW), lambda b, k, q: (b, 0, 0)),
            pl.BlockSpec((1, C, HW), lambda b, k, q: (b, 0, 0)),
            pl.BlockSpec((HW, 1), lambda b, k, q: (0, 0)),
            pl.BlockSpec((1, HW), lambda b, k, q: (0, 0)),
        ],
        out_specs=pl.BlockSpec((1, 1, HW), lambda b, k, q: (b, 0, 0)),
        out_shape=jax.ShapeDtypeStruct((B, 1, HW), jnp.int32),
        scratch_shapes=[
            pltpu.VMEM((C, _KT), jnp.float32),
            pltpu.VMEM((1, HW), jnp.float32),
            pltpu.VMEM((1, HW), jnp.int32),
        ],
        compiler_params=pltpu.CompilerParams(
            dimension_semantics=("parallel", "arbitrary", "arbitrary"),
        ),
    )(xin, rin, maskk, maskq)


def _sc_gather(table, idx):
    nrows, C = table.shape
    rows_per_w = nrows // _NW
    chunk = min(rows_per_w, 256)

    mesh = plsc.VectorSubcoreMesh(core_axis_name="c", subcore_axis_name="s")

    @functools.partial(
        pl.kernel,
        mesh=mesh,
        out_type=jax.ShapeDtypeStruct((nrows, C), jnp.float32),
        scratch_types=[
            pltpu.VMEM((rows_per_w,), jnp.int32),
            pltpu.VMEM((chunk, C), jnp.float32),
            pltpu.SemaphoreType.DMA,
        ],
    )
    def gather_k(table_hbm, idx_hbm, out_hbm, idx_v, buf, sem):
        wid = lax.axis_index("s") * _NC + lax.axis_index("c")
        base = wid * rows_per_w
        pltpu.sync_copy(idx_hbm.at[pl.ds(base, rows_per_w)], idx_v)
        for c in range(rows_per_w // chunk):
            src = table_hbm.at[idx_v.at[pl.ds(c * chunk, chunk)]]
            pltpu.async_copy(src, buf, sem).wait()
            pltpu.sync_copy(buf, out_hbm.at[pl.ds(base + c * chunk, chunk)])

    return gather_k(table, idx)


def kernel(input, ref, mask):
    B, C, H, W = input.shape
    HW = H * W
    xin = input.reshape(B, C, HW)
    rin = ref.reshape(B, C, HW)
    mflat = mask.reshape(HW).astype(jnp.int32)
    maskk = mflat.reshape(HW, 1)
    maskq = mflat.reshape(1, HW)

    ind = _argmax_indices(xin, rin, maskk, maskq)
    table = xin.transpose(0, 2, 1).reshape(B * HW, C)
    rows = _sc_gather(table, ind.reshape(B * HW))
    return rows.reshape(B, HW, C).transpose(0, 2, 1).reshape(B, C, H, W)
